# baseline (speedup 1.0000x reference)
import jax
import jax.numpy as jnp
from jax import lax
from jax.experimental import pallas as pl
from jax.experimental.pallas import tpu as pltpu
from jax.experimental.pallas import tpu_sc as plsc

N_ATOMS = 50000
N_EDGES = 800000
FEA = 64
N_CONV = 4
N_GRAPHS = 16

EB = 128
NB = 2
K = NB * EB
ROWS_PT = 392
CHUNKS = ROWS_PT // NB
E_PAD = 16 * ROWS_PT * EB
N_PAD = 50048
ROWS_N_PT = N_PAD // 16
GW = 16
TW = 32
OP = 512
NOP = ROWS_N_PT // OP
OPT = ROWS_N_PT - NOP * OP


def _conv_sc_body(src_hbm, dst_hbm, p_hbm, q_hbm, b_hbm, out_hbm,
                  sidxr, didxr, sidx2, didx2g,
                  pbuf, qbuf, bbuf, mbuf, agg_sh, gsem, ssem):
    c = lax.axis_index("c")
    s = lax.axis_index("s")
    n0 = pl.multiple_of(s * ROWS_N_PT, 8)
    z16 = jnp.zeros((16,), jnp.float32)

    def idx_rows(j):
        return pl.multiple_of(s * ROWS_PT + j * NB, 2)

    def load_idx(j, par):
        r = idx_rows(j)
        pltpu.sync_copy(src_hbm.at[pl.ds(r, NB)],
                        sidxr.at[pl.ds(par * NB, NB)])
        pltpu.sync_copy(dst_hbm.at[pl.ds(r, NB)],
                        didxr.at[pl.ds(par * NB, NB)])

    def repack(par, off_p, qmax):
        for k in range(NB):
            for t in range(EB // 16):
                sl = pl.ds(t * 16, 16)
                s_v = sidxr[par * NB + k, sl]
                d_v = didxr[par * NB + k, sl]
                sidx2[par * NB + k, sl] = s_v + off_p
                didx2g[par * NB + k, sl] = jnp.minimum(d_v + off_p, qmax)

    def issue_gathers(j, par, g):
        for k in range(NB):
            pltpu.async_copy(p_hbm.at[sidx2.at[par * NB + k]],
                             pbuf.at[pl.ds(par * K + k * EB, EB)], gsem)
            pltpu.async_copy(q_hbm.at[didx2g.at[par * NB + k]],
                             qbuf.at[pl.ds(par * K + k * EB, EB)], gsem)
        e0 = pl.multiple_of(g * E_PAD + (s * ROWS_PT + j * NB) * EB, 8)
        pltpu.async_copy(b_hbm.at[pl.ds(e0, K)],
                         bbuf.at[pl.ds(par * K, K)], gsem)

    def drain_gathers(par):
        pltpu.make_async_copy(p_hbm.at[pl.ds(0, K)],
                              pbuf.at[pl.ds(par * K, K)], gsem).wait()
        pltpu.make_async_copy(q_hbm.at[pl.ds(0, K)],
                              qbuf.at[pl.ds(par * K, K)], gsem).wait()
        pltpu.make_async_copy(b_hbm.at[pl.ds(0, K)],
                              bbuf.at[pl.ds(par * K, K)], gsem).wait()

    def issue_scatters(par):
        for k in range(NB):
            pltpu.async_copy(mbuf.at[pl.ds(par * K + k * EB, EB)],
                             agg_sh.at[didxr.at[par * NB + k]], ssem,
                             add=True)

    def drain_scatters(par):
        for k in range(NB):
            pltpu.make_async_copy(mbuf.at[pl.ds(par * K + k * EB, EB)],
                                  agg_sh.at[didxr.at[par * NB + k]],
                                  ssem).wait()

    def compute(par):
        @plsc.parallel_loop(par * K, (par + 1) * K, step=1, unroll=4)
        def _row(r):
            slo = pl.ds(0, GW)
            shi = pl.ds(GW, GW)
            cv = pbuf[r, slo] + qbuf[r, slo] + bbuf[r, slo]
            gv = pbuf[r, shi] + qbuf[r, shi] + bbuf[r, shi]
            denom = (1.0 + jnp.exp(-cv)) * (1.0 + jnp.exp(-gv))
            mbuf[r, slo] = cv / denom

    def pass_body(p):
        g = 2 * c + p
        off_p = g * N_ATOMS
        qmax = 4 * N_ATOMS - 1

        def zrow(r, carry):
            mbuf[r, pl.ds(0, GW)] = z16
            return carry

        lax.fori_loop(0, OP, zrow, 0)

        def zpiece(i, carry):
            pltpu.sync_copy(mbuf, agg_sh.at[pl.ds(n0 + i * OP, OP)])
            return carry

        lax.fori_loop(0, NOP, zpiece, 0)
        pltpu.sync_copy(mbuf.at[pl.ds(0, OPT)],
                        agg_sh.at[pl.ds(n0 + NOP * OP, OPT)])
        plsc.subcore_barrier()

        load_idx(0, 0)
        repack(0, off_p, qmax)
        issue_gathers(0, 0, g)

        def super_chunk(u, carry):
            for par in range(2):
                j = 2 * u + par

                if par == 0:
                    @pl.when(u > 0)
                    def _():
                        drain_scatters(1)
                else:
                    drain_scatters(0)

                @pl.when(j + 1 < CHUNKS)
                def _():
                    load_idx(j + 1, 1 - par)
                    repack(1 - par, off_p, qmax)
                    issue_gathers(j + 1, 1 - par, g)

                drain_gathers(par)
                compute(par)
                issue_scatters(par)
            return carry

        lax.fori_loop(0, CHUNKS // 2, super_chunk, 0)
        drain_scatters(1)
        plsc.subcore_barrier()

        def opiece(i, carry):
            r = pl.multiple_of(n0 + i * OP, 8)
            pltpu.sync_copy(
                agg_sh.at[pl.ds(r, OP)],
                out_hbm.at[pl.ds(pl.multiple_of(g * N_PAD + r, 8), OP)])
            return carry

        lax.fori_loop(0, NOP, opiece, 0)
        rt = pl.multiple_of(n0 + NOP * OP, 8)
        pltpu.sync_copy(
            agg_sh.at[pl.ds(rt, OPT)],
            out_hbm.at[pl.ds(pl.multiple_of(g * N_PAD + rt, 8), OPT)])
        plsc.subcore_barrier()

    for p in range(2):
        pass_body(p)


_conv_sc = pl.kernel(
    _conv_sc_body,
    out_type=jax.ShapeDtypeStruct((4 * N_PAD, GW), jnp.float32),
    mesh=plsc.VectorSubcoreMesh(core_axis_name="c", subcore_axis_name="s"),
    scratch_types=[
        pltpu.VMEM((2 * NB, EB), jnp.int32),
        pltpu.VMEM((2 * NB, EB), jnp.int32),
        pltpu.VMEM((2 * NB, EB), jnp.int32),
        pltpu.VMEM((2 * NB, EB), jnp.int32),
        pltpu.VMEM((2 * K, TW), jnp.float32),
        pltpu.VMEM((2 * K, TW), jnp.float32),
        pltpu.VMEM((2 * K, TW), jnp.float32),
        pltpu.VMEM((2 * K, GW), jnp.float32),
        pltpu.VMEM_SHARED((N_PAD, GW), jnp.float32),
        pltpu.SemaphoreType.DMA,
        pltpu.SemaphoreType.DMA,
    ],
    compiler_params=pltpu.CompilerParams(use_tc_tiling_on_sc=False),
)


def _layernorm(x):
    m = jnp.mean(x, axis=-1, keepdims=True)
    v = jnp.var(x, axis=-1, keepdims=True)
    return (x - m) / jnp.sqrt(v + 1e-5)


EBLK = 2048
NBLK = 2000


def _bterm_body(bb_ref, wb_ref, wall_ref, bc_ref, o0, o1, o2, o3):
    y = jnp.dot(bb_ref[...], wb_ref[...], preferred_element_type=jnp.float32)
    bf = y * jax.nn.sigmoid(y)
    z = jnp.dot(bf, wall_ref[...],
                preferred_element_type=jnp.float32) + bc_ref[...]
    for i, o in enumerate((o0, o1, o2, o3)):
        for g in range(4):
            col = (i * 4 + g) * TW
            o[g] = z[:, col:col + TW]


def _bterms(bb_p, Wb, WB, bcat):
    wall = jnp.concatenate(
        [WB[i, g] for i in range(N_CONV) for g in range(4)], 1)
    bc = jnp.concatenate(
        [bcat[i, g] for i in range(N_CONV) for g in range(4)]).reshape(1, -1)
    outs = pl.pallas_call(
        _bterm_body,
        grid=(E_PAD // EBLK,),
        in_specs=[
            pl.BlockSpec((EBLK, 31), lambda e: (e, 0)),
            pl.BlockSpec((31, FEA), lambda e: (0, 0)),
            pl.BlockSpec((FEA, 16 * TW), lambda e: (0, 0)),
            pl.BlockSpec((1, 16 * TW), lambda e: (0, 0)),
        ],
        out_specs=[pl.BlockSpec((4, EBLK, TW), lambda e: (0, e, 0))] * 4,
        out_shape=[jax.ShapeDtypeStruct((4, E_PAD, TW), jnp.float32)] * 4,
    )(bb_p, Wb, wall, bc)
    return [o.reshape(4 * E_PAD, TW) for o in outs]


def _embed_body(an_ref, emb_ref, x_ref):
    an = an_ref[0, 0, :]
    onehot = (an[:, None] == lax.broadcasted_iota(jnp.int32, (NBLK, 128), 1))
    x_ref[...] = jnp.dot(onehot.astype(jnp.float32), emb_ref[...],
                         preferred_element_type=jnp.float32)


def _embed(an3, emb_p):
    return pl.pallas_call(
        _embed_body,
        grid=(N_ATOMS // NBLK,),
        in_specs=[
            pl.BlockSpec((1, 1, NBLK), lambda b: (b, 0, 0)),
            pl.BlockSpec((128, FEA), lambda b: (0, 0)),
        ],
        out_specs=pl.BlockSpec((NBLK, FEA), lambda b: (b, 0)),
        out_shape=jax.ShapeDtypeStruct((N_ATOMS, FEA), jnp.float32),
    )(an3, emb_p)


def _proj_body(x_ref, w_ref, p_ref, q_ref):
    z = jnp.dot(x_ref[...], w_ref[...], preferred_element_type=jnp.float32)
    for g in range(4):
        p_ref[g] = z[:, g * TW:(g + 1) * TW]
        q_ref[g] = z[:, (4 + g) * TW:(5 + g) * TW]


def _proj(x, Wpq):
    P, Q = pl.pallas_call(
        _proj_body,
        grid=(N_ATOMS // NBLK,),
        in_specs=[
            pl.BlockSpec((NBLK, FEA), lambda b: (b, 0)),
            pl.BlockSpec((FEA, 8 * TW), lambda b: (0, 0)),
        ],
        out_specs=[pl.BlockSpec((4, NBLK, TW), lambda b: (0, b, 0))] * 2,
        out_shape=[jax.ShapeDtypeStruct((4, N_ATOMS, TW), jnp.float32)] * 2,
    )(x, Wpq)
    return P.reshape(4 * N_ATOMS, TW), Q.reshape(4 * N_ATOMS, TW)


def _update_body(x_ref, agg_ref, s_ref, b_ref, xo_ref):
    a = jnp.concatenate([agg_ref[g] for g in range(4)], axis=-1)
    xo_ref[...] = x_ref[...] + _layernorm(a) * s_ref[...] + b_ref[...]


def _update(x, agg4, ln_s_i, ln_b_i):
    return pl.pallas_call(
        _update_body,
        grid=(N_ATOMS // NBLK,),
        in_specs=[
            pl.BlockSpec((NBLK, FEA), lambda b: (b, 0)),
            pl.BlockSpec((4, NBLK, GW), lambda b: (0, b, 0)),
            pl.BlockSpec((1, FEA), lambda b: (0, 0)),
            pl.BlockSpec((1, FEA), lambda b: (0, 0)),
        ],
        out_specs=pl.BlockSpec((NBLK, FEA), lambda b: (b, 0)),
        out_shape=jax.ShapeDtypeStruct((N_ATOMS, FEA), jnp.float32),
    )(x, agg4, ln_s_i, ln_b_i)


def _readout_body(ow_ref, x_ref, wm_ref, bm_ref, wo_ref, bo_ref,
                  e_ref, gsum_ref, gcnt_ref):
    b = pl.program_id(0)

    @pl.when(b == 0)
    def _():
        gsum_ref[...] = jnp.zeros_like(gsum_ref)
        gcnt_ref[...] = jnp.zeros_like(gcnt_ref)

    ow = ow_ref[0, 0, :]
    onehot = (ow[:, None] == lax.broadcasted_iota(
        jnp.int32, (NBLK, N_GRAPHS), 1)).astype(jnp.float32)
    gsum_ref[...] += jnp.dot(onehot.T, x_ref[...],
                             preferred_element_type=jnp.float32)
    gcnt_ref[...] += jnp.sum(onehot, axis=0, keepdims=True)

    @pl.when(b == N_ATOMS // NBLK - 1)
    def _():
        h = gsum_ref[...] / jnp.maximum(gcnt_ref[...], 1.0).T
        for j in range(3):
            y = jnp.dot(h, wm_ref[j], preferred_element_type=jnp.float32) \
                + bm_ref[j]
            h = y * jax.nn.sigmoid(y)
        e_ref[...] = jnp.dot(h, wo_ref[...],
                             preferred_element_type=jnp.float32) + bo_ref[...]


def _readout(ow3, x, Wm, bm2, Wout, bout2):
    e, _, _ = pl.pallas_call(
        _readout_body,
        grid=(N_ATOMS // NBLK,),
        in_specs=[
            pl.BlockSpec((1, 1, NBLK), lambda b: (b, 0, 0)),
            pl.BlockSpec((NBLK, FEA), lambda b: (b, 0)),
            pl.BlockSpec((3, FEA, FEA), lambda b: (0, 0, 0)),
            pl.BlockSpec((3, 1, FEA), lambda b: (0, 0, 0)),
            pl.BlockSpec((FEA, 1), lambda b: (0, 0)),
            pl.BlockSpec((1, 1), lambda b: (0, 0)),
        ],
        out_specs=[
            pl.BlockSpec((N_GRAPHS, 1), lambda b: (0, 0)),
            pl.BlockSpec((N_GRAPHS, FEA), lambda b: (0, 0)),
            pl.BlockSpec((1, N_GRAPHS), lambda b: (0, 0)),
        ],
        out_shape=[
            jax.ShapeDtypeStruct((N_GRAPHS, 1), jnp.float32),
            jax.ShapeDtypeStruct((N_GRAPHS, FEA), jnp.float32),
            jax.ShapeDtypeStruct((1, N_GRAPHS), jnp.float32),
        ],
    )(ow3, x, Wm, bm2, Wout, bout2)
    return e[:, 0]


def kernel(atomic_numbers, bond_bases_ag, batched_atom_graph, atom_owners,
           emb, Wb, W1, b1, Wg, bg, ln_s, ln_b, Wm, bm, Wout, bout):
    n = N_ATOMS
    e = N_EDGES

    src = batched_atom_graph[:, 0].astype(jnp.int32)
    dst = batched_atom_graph[:, 1].astype(jnp.int32)
    pad = E_PAD - e
    src_p = jnp.concatenate(
        [src, jnp.zeros((pad,), jnp.int32)]).reshape(E_PAD // EB, EB)
    dst_p = jnp.concatenate(
        [dst, jnp.full((pad,), N_ATOMS, jnp.int32)]).reshape(E_PAD // EB, EB)
    bb_p = jnp.concatenate(
        [bond_bases_ag, jnp.zeros((pad, bond_bases_ag.shape[1]), jnp.float32)])

    def pack(lo):
        return jnp.stack([
            jnp.concatenate([W1[i][lo:lo + FEA, g * GW:(g + 1) * GW],
                             Wg[i][lo:lo + FEA, g * GW:(g + 1) * GW]], 1)
            for i in range(N_CONV) for g in range(4)]).reshape(
                N_CONV, 4, FEA, TW)

    WP = pack(0)
    WQ = pack(FEA)
    WB = pack(2 * FEA)
    bcat = jnp.stack([
        jnp.concatenate([b1[i][g * GW:(g + 1) * GW],
                         bg[i][g * GW:(g + 1) * GW]])
        for i in range(N_CONV) for g in range(4)]).reshape(N_CONV, 4, TW)

    an3 = atomic_numbers.astype(jnp.int32).reshape(
        N_ATOMS // NBLK, 1, NBLK)
    emb_p = jnp.concatenate(
        [emb, jnp.zeros((128 - emb.shape[0], FEA), jnp.float32)], 0)
    x = _embed(an3, emb_p)
    Bterms = _bterms(bb_p, Wb, WB, bcat)

    Wpq = [jnp.concatenate([WP[i, g] for g in range(4)]
                           + [WQ[i, g] for g in range(4)], 1)
           for i in range(N_CONV)]

    for i in range(N_CONV):
        P, Q = _proj(x, Wpq[i])
        agg4 = _conv_sc(src_p, dst_p, P, Q, Bterms[i])
        agg4 = agg4.reshape(4, N_PAD, GW)
        x = _update(x, agg4, ln_s[i].reshape(1, FEA),
                    ln_b[i].reshape(1, FEA))

    ow3 = atom_owners.astype(jnp.int32).reshape(N_ATOMS // NBLK, 1, NBLK)
    return _readout(ow3, x, Wm, bm.reshape(3, 1, FEA), Wout,
                    bout.reshape(1, 1))


# speedup vs baseline: 1.5563x; 1.5563x over previous
import jax
import jax.numpy as jnp
from jax import lax
from jax.experimental import pallas as pl
from jax.experimental.pallas import tpu as pltpu
from jax.experimental.pallas import tpu_sc as plsc

N_ATOMS = 50000
N_EDGES = 800000
FEA = 64
N_CONV = 4
N_GRAPHS = 16

EB = 128
NB = 2
K = NB * EB
ROWS_PT = 392
CHUNKS = ROWS_PT // NB
E_PAD = 16 * ROWS_PT * EB
N_PAD = 50048
ROWS_N_PT = N_PAD // 16
GW = 16
TW = 32
OP = 512
NOP = ROWS_N_PT // OP
OPT = ROWS_N_PT - NOP * OP


def _conv_sc_body(src_hbm, dst_hbm, p_hbm, q_hbm, b_hbm, out_hbm,
                  sidxr, didxr, sidx2, didx2g,
                  pbuf, qbuf, bbuf, mbuf, agg_sh, gsem, ssem):
    c = lax.axis_index("c")
    s = lax.axis_index("s")
    n0 = pl.multiple_of(s * ROWS_N_PT, 8)
    z16 = jnp.zeros((16,), jnp.float32)

    def idx_rows(j):
        return pl.multiple_of(s * ROWS_PT + j * NB, 2)

    def load_idx(j, par):
        r = idx_rows(j)
        pltpu.sync_copy(src_hbm.at[pl.ds(r, NB)],
                        sidxr.at[pl.ds(par * NB, NB)])
        pltpu.sync_copy(dst_hbm.at[pl.ds(r, NB)],
                        didxr.at[pl.ds(par * NB, NB)])

    def repack(par, off_p, qmax):
        for k in range(NB):
            for t in range(EB // 16):
                sl = pl.ds(t * 16, 16)
                s_v = sidxr[par * NB + k, sl]
                d_v = didxr[par * NB + k, sl]
                sidx2[par * NB + k, sl] = s_v + off_p
                didx2g[par * NB + k, sl] = jnp.minimum(d_v + off_p, qmax)

    def issue_gathers(j, par, g):
        for k in range(NB):
            pltpu.async_copy(p_hbm.at[sidx2.at[par * NB + k]],
                             pbuf.at[pl.ds(par * K + k * EB, EB)], gsem)
            pltpu.async_copy(q_hbm.at[didx2g.at[par * NB + k]],
                             qbuf.at[pl.ds(par * K + k * EB, EB)], gsem)
        e0 = pl.multiple_of((s * ROWS_PT + j * NB) * EB, 8)
        pltpu.async_copy(b_hbm.at[pl.ds(e0, K),
                                  pl.ds(pl.multiple_of(g * TW, 32), TW)],
                         bbuf.at[pl.ds(par * K, K)], gsem)

    def drain_gathers(par):
        pltpu.make_async_copy(p_hbm.at[pl.ds(0, K)],
                              pbuf.at[pl.ds(par * K, K)], gsem).wait()
        pltpu.make_async_copy(q_hbm.at[pl.ds(0, K)],
                              qbuf.at[pl.ds(par * K, K)], gsem).wait()
        pltpu.make_async_copy(b_hbm.at[pl.ds(0, K), pl.ds(0, TW)],
                              bbuf.at[pl.ds(par * K, K)], gsem).wait()

    def issue_scatters(par):
        for k in range(NB):
            pltpu.async_copy(mbuf.at[pl.ds(par * K + k * EB, EB)],
                             agg_sh.at[didxr.at[par * NB + k]], ssem,
                             add=True)

    def drain_scatters(par):
        for k in range(NB):
            pltpu.make_async_copy(mbuf.at[pl.ds(par * K + k * EB, EB)],
                                  agg_sh.at[didxr.at[par * NB + k]],
                                  ssem).wait()

    def compute(par):
        @plsc.parallel_loop(par * K, (par + 1) * K, step=1, unroll=4)
        def _row(r):
            slo = pl.ds(0, GW)
            shi = pl.ds(GW, GW)
            cv = pbuf[r, slo] + qbuf[r, slo] + bbuf[r, slo]
            gv = pbuf[r, shi] + qbuf[r, shi] + bbuf[r, shi]
            denom = (1.0 + jnp.exp(-cv)) * (1.0 + jnp.exp(-gv))
            mbuf[r, slo] = cv / denom

    def pass_body(p):
        g = 2 * c + p
        off_p = g * N_ATOMS
        qmax = 4 * N_ATOMS - 1

        def zrow(r, carry):
            mbuf[r, pl.ds(0, GW)] = z16
            return carry

        lax.fori_loop(0, OP, zrow, 0)

        def zpiece(i, carry):
            pltpu.sync_copy(mbuf, agg_sh.at[pl.ds(n0 + i * OP, OP)])
            return carry

        lax.fori_loop(0, NOP, zpiece, 0)
        pltpu.sync_copy(mbuf.at[pl.ds(0, OPT)],
                        agg_sh.at[pl.ds(n0 + NOP * OP, OPT)])
        plsc.subcore_barrier()

        load_idx(0, 0)
        repack(0, off_p, qmax)
        issue_gathers(0, 0, g)

        def super_chunk(u, carry):
            for par in range(2):
                j = 2 * u + par

                if par == 0:
                    @pl.when(u > 0)
                    def _():
                        drain_scatters(1)
                else:
                    drain_scatters(0)

                @pl.when(j + 1 < CHUNKS)
                def _():
                    load_idx(j + 1, 1 - par)
                    repack(1 - par, off_p, qmax)
                    issue_gathers(j + 1, 1 - par, g)

                drain_gathers(par)
                compute(par)
                issue_scatters(par)
            return carry

        lax.fori_loop(0, CHUNKS // 2, super_chunk, 0)
        drain_scatters(1)
        plsc.subcore_barrier()

        def opiece(i, carry):
            r = pl.multiple_of(n0 + i * OP, 8)
            pltpu.sync_copy(
                agg_sh.at[pl.ds(r, OP)],
                out_hbm.at[pl.ds(pl.multiple_of(g * N_PAD + r, 8), OP)])
            return carry

        lax.fori_loop(0, NOP, opiece, 0)
        rt = pl.multiple_of(n0 + NOP * OP, 8)
        pltpu.sync_copy(
            agg_sh.at[pl.ds(rt, OPT)],
            out_hbm.at[pl.ds(pl.multiple_of(g * N_PAD + rt, 8), OPT)])
        plsc.subcore_barrier()

    for p in range(2):
        pass_body(p)


_conv_sc = pl.kernel(
    _conv_sc_body,
    out_type=jax.ShapeDtypeStruct((4 * N_PAD, GW), jnp.float32),
    mesh=plsc.VectorSubcoreMesh(core_axis_name="c", subcore_axis_name="s"),
    scratch_types=[
        pltpu.VMEM((2 * NB, EB), jnp.int32),
        pltpu.VMEM((2 * NB, EB), jnp.int32),
        pltpu.VMEM((2 * NB, EB), jnp.int32),
        pltpu.VMEM((2 * NB, EB), jnp.int32),
        pltpu.VMEM((2 * K, TW), jnp.float32),
        pltpu.VMEM((2 * K, TW), jnp.float32),
        pltpu.VMEM((2 * K, TW), jnp.float32),
        pltpu.VMEM((2 * K, GW), jnp.float32),
        pltpu.VMEM_SHARED((N_PAD, GW), jnp.float32),
        pltpu.SemaphoreType.DMA,
        pltpu.SemaphoreType.DMA,
    ],
    compiler_params=pltpu.CompilerParams(use_tc_tiling_on_sc=False),
)


def _layernorm(x):
    m = jnp.mean(x, axis=-1, keepdims=True)
    v = jnp.var(x, axis=-1, keepdims=True)
    return (x - m) / jnp.sqrt(v + 1e-5)


EBLK = 2048
NBLK = 2000


def _bterm_body(bb_ref, wb_ref, wall_ref, bc_ref, o0, o1, o2, o3):
    y = jnp.dot(bb_ref[...], wb_ref[...], preferred_element_type=jnp.float32)
    bf = y * jax.nn.sigmoid(y)
    z = jnp.dot(bf, wall_ref[...],
                preferred_element_type=jnp.float32) + bc_ref[...]
    for i, o in enumerate((o0, o1, o2, o3)):
        o[...] = z[:, i * 128:(i + 1) * 128]


def _bterms(bb_p, Wb, WB, bcat):
    wall = jnp.concatenate(
        [WB[i, g] for i in range(N_CONV) for g in range(4)], 1)
    bc = jnp.concatenate(
        [bcat[i, g] for i in range(N_CONV) for g in range(4)]).reshape(1, -1)
    outs = pl.pallas_call(
        _bterm_body,
        grid=(E_PAD // EBLK,),
        in_specs=[
            pl.BlockSpec((EBLK, 31), lambda e: (e, 0)),
            pl.BlockSpec((31, FEA), lambda e: (0, 0)),
            pl.BlockSpec((FEA, 16 * TW), lambda e: (0, 0)),
            pl.BlockSpec((1, 16 * TW), lambda e: (0, 0)),
        ],
        out_specs=[pl.BlockSpec((EBLK, 128), lambda e: (e, 0))] * 4,
        out_shape=[jax.ShapeDtypeStruct((E_PAD, 128), jnp.float32)] * 4,
    )(bb_p, Wb, wall, bc)
    return outs


def _embed_body(an_ref, emb_ref, x_ref):
    an = an_ref[0, 0, :]
    onehot = (an[:, None] == lax.broadcasted_iota(jnp.int32, (NBLK, 128), 1))
    x_ref[...] = jnp.dot(onehot.astype(jnp.float32), emb_ref[...],
                         preferred_element_type=jnp.float32)


def _embed(an3, emb_p):
    return pl.pallas_call(
        _embed_body,
        grid=(N_ATOMS // NBLK,),
        in_specs=[
            pl.BlockSpec((1, 1, NBLK), lambda b: (b, 0, 0)),
            pl.BlockSpec((128, FEA), lambda b: (0, 0)),
        ],
        out_specs=pl.BlockSpec((NBLK, FEA), lambda b: (b, 0)),
        out_shape=jax.ShapeDtypeStruct((N_ATOMS, FEA), jnp.float32),
    )(an3, emb_p)


def _proj_body(x_ref, wp_ref, wq_ref, p_ref, q_ref):
    x = x_ref[...]
    for g in range(4):
        p_ref[g] = jnp.dot(x, wp_ref[g], preferred_element_type=jnp.float32)
        q_ref[g] = jnp.dot(x, wq_ref[g], preferred_element_type=jnp.float32)


def _proj(x, WPi, WQi):
    P, Q = pl.pallas_call(
        _proj_body,
        grid=(N_ATOMS // NBLK,),
        in_specs=[
            pl.BlockSpec((NBLK, FEA), lambda b: (b, 0)),
            pl.BlockSpec((4, FEA, TW), lambda b: (0, 0, 0)),
            pl.BlockSpec((4, FEA, TW), lambda b: (0, 0, 0)),
        ],
        out_specs=[pl.BlockSpec((4, NBLK, TW), lambda b: (0, b, 0))] * 2,
        out_shape=[jax.ShapeDtypeStruct((4, N_ATOMS, TW), jnp.float32)] * 2,
    )(x, WPi, WQi)
    return P.reshape(4 * N_ATOMS, TW), Q.reshape(4 * N_ATOMS, TW)


def _update_body(x_ref, agg_ref, s_ref, b_ref, xo_ref):
    a = jnp.concatenate([agg_ref[g] for g in range(4)], axis=-1)
    xo_ref[...] = x_ref[...] + _layernorm(a) * s_ref[...] + b_ref[...]


def _update(x, agg4, ln_s_i, ln_b_i):
    return pl.pallas_call(
        _update_body,
        grid=(N_ATOMS // NBLK,),
        in_specs=[
            pl.BlockSpec((NBLK, FEA), lambda b: (b, 0)),
            pl.BlockSpec((4, NBLK, GW), lambda b: (0, b, 0)),
            pl.BlockSpec((1, FEA), lambda b: (0, 0)),
            pl.BlockSpec((1, FEA), lambda b: (0, 0)),
        ],
        out_specs=pl.BlockSpec((NBLK, FEA), lambda b: (b, 0)),
        out_shape=jax.ShapeDtypeStruct((N_ATOMS, FEA), jnp.float32),
    )(x, agg4, ln_s_i, ln_b_i)


def _readout_body(ow_ref, x_ref, wm_ref, bm_ref, wo_ref, bo_ref,
                  e_ref, gsum_ref, gcnt_ref):
    b = pl.program_id(0)

    @pl.when(b == 0)
    def _():
        gsum_ref[...] = jnp.zeros_like(gsum_ref)
        gcnt_ref[...] = jnp.zeros_like(gcnt_ref)

    ow = ow_ref[0, 0, :]
    onehot = (ow[:, None] == lax.broadcasted_iota(
        jnp.int32, (NBLK, N_GRAPHS), 1)).astype(jnp.float32)
    gsum_ref[...] += jnp.dot(onehot.T, x_ref[...],
                             preferred_element_type=jnp.float32)
    gcnt_ref[...] += jnp.sum(onehot, axis=0, keepdims=True)

    @pl.when(b == N_ATOMS // NBLK - 1)
    def _():
        h = gsum_ref[...] / jnp.maximum(gcnt_ref[...], 1.0).T
        for j in range(3):
            y = jnp.dot(h, wm_ref[j], preferred_element_type=jnp.float32) \
                + bm_ref[j]
            h = y * jax.nn.sigmoid(y)
        e_ref[...] = jnp.dot(h, wo_ref[...],
                             preferred_element_type=jnp.float32) + bo_ref[...]


def _readout(ow3, x, Wm, bm2, Wout, bout2):
    e, _, _ = pl.pallas_call(
        _readout_body,
        grid=(N_ATOMS // NBLK,),
        in_specs=[
            pl.BlockSpec((1, 1, NBLK), lambda b: (b, 0, 0)),
            pl.BlockSpec((NBLK, FEA), lambda b: (b, 0)),
            pl.BlockSpec((3, FEA, FEA), lambda b: (0, 0, 0)),
            pl.BlockSpec((3, 1, FEA), lambda b: (0, 0, 0)),
            pl.BlockSpec((FEA, 1), lambda b: (0, 0)),
            pl.BlockSpec((1, 1), lambda b: (0, 0)),
        ],
        out_specs=[
            pl.BlockSpec((N_GRAPHS, 1), lambda b: (0, 0)),
            pl.BlockSpec((N_GRAPHS, FEA), lambda b: (0, 0)),
            pl.BlockSpec((1, N_GRAPHS), lambda b: (0, 0)),
        ],
        out_shape=[
            jax.ShapeDtypeStruct((N_GRAPHS, 1), jnp.float32),
            jax.ShapeDtypeStruct((N_GRAPHS, FEA), jnp.float32),
            jax.ShapeDtypeStruct((1, N_GRAPHS), jnp.float32),
        ],
    )(ow3, x, Wm, bm2, Wout, bout2)
    return e[:, 0]


def kernel(atomic_numbers, bond_bases_ag, batched_atom_graph, atom_owners,
           emb, Wb, W1, b1, Wg, bg, ln_s, ln_b, Wm, bm, Wout, bout):
    n = N_ATOMS
    e = N_EDGES

    src = batched_atom_graph[:, 0].astype(jnp.int32)
    dst = batched_atom_graph[:, 1].astype(jnp.int32)
    pad = E_PAD - e
    src_p = jnp.concatenate(
        [src, jnp.zeros((pad,), jnp.int32)]).reshape(E_PAD // EB, EB)
    dst_p = jnp.concatenate(
        [dst, jnp.full((pad,), N_ATOMS, jnp.int32)]).reshape(E_PAD // EB, EB)
    bb_p = jnp.concatenate(
        [bond_bases_ag, jnp.zeros((pad, bond_bases_ag.shape[1]), jnp.float32)])

    def pack(lo):
        return jnp.stack([
            jnp.concatenate([W1[i][lo:lo + FEA, g * GW:(g + 1) * GW],
                             Wg[i][lo:lo + FEA, g * GW:(g + 1) * GW]], 1)
            for i in range(N_CONV) for g in range(4)]).reshape(
                N_CONV, 4, FEA, TW)

    WP = pack(0)
    WQ = pack(FEA)
    WB = pack(2 * FEA)
    bcat = jnp.stack([
        jnp.concatenate([b1[i][g * GW:(g + 1) * GW],
                         bg[i][g * GW:(g + 1) * GW]])
        for i in range(N_CONV) for g in range(4)]).reshape(N_CONV, 4, TW)

    an3 = atomic_numbers.astype(jnp.int32).reshape(
        N_ATOMS // NBLK, 1, NBLK)
    emb_p = jnp.concatenate(
        [emb, jnp.zeros((128 - emb.shape[0], FEA), jnp.float32)], 0)
    x = _embed(an3, emb_p)
    Bterms = _bterms(bb_p, Wb, WB, bcat)

    for i in range(N_CONV):
        P, Q = _proj(x, WP[i], WQ[i])
        agg4 = _conv_sc(src_p, dst_p, P, Q, Bterms[i])
        agg4 = agg4.reshape(4, N_PAD, GW)
        x = _update(x, agg4, ln_s[i].reshape(1, FEA),
                    ln_b[i].reshape(1, FEA))

    ow3 = atom_owners.astype(jnp.int32).reshape(N_ATOMS // NBLK, 1, NBLK)
    return _readout(ow3, x, Wm, bm.reshape(3, 1, FEA), Wout,
                    bout.reshape(1, 1))


# speedup vs baseline: 1.5806x; 1.0156x over previous
import jax
import jax.numpy as jnp
from jax import lax
from jax.experimental import pallas as pl
from jax.experimental.pallas import tpu as pltpu
from jax.experimental.pallas import tpu_sc as plsc

N_ATOMS = 50000
N_EDGES = 800000
FEA = 64
N_CONV = 4
N_GRAPHS = 16

EB = 128
NB = 2
K = NB * EB
ROWS_PT = 392
CHUNKS = ROWS_PT // NB
E_PAD = 16 * ROWS_PT * EB
N_PAD = 50048
ROWS_N_PT = N_PAD // 16
GW = 16
TW = 32
OP = 512
NOP = ROWS_N_PT // OP
OPT = ROWS_N_PT - NOP * OP


def _conv_sc_body(src_hbm, dst_hbm, p_hbm, q_hbm, b_hbm, out_hbm,
                  sidxr, didxr, sidx2, didx2g,
                  pbuf, qbuf, bbuf, mbuf, agg_sh, gsem, ssem):
    c = lax.axis_index("c")
    s = lax.axis_index("s")
    n0 = pl.multiple_of(s * ROWS_N_PT, 8)
    z16 = jnp.zeros((16,), jnp.float32)

    def idx_rows(j):
        return pl.multiple_of(s * ROWS_PT + j * NB, 2)

    def load_idx(j, par):
        r = idx_rows(j)
        pltpu.sync_copy(src_hbm.at[pl.ds(r, NB)],
                        sidxr.at[pl.ds(par * NB, NB)])
        pltpu.sync_copy(dst_hbm.at[pl.ds(r, NB)],
                        didxr.at[pl.ds(par * NB, NB)])

    def repack(par, off_p, qmax):
        for k in range(NB):
            for t in range(EB // 16):
                sl = pl.ds(t * 16, 16)
                s_v = sidxr[par * NB + k, sl]
                d_v = didxr[par * NB + k, sl]
                sidx2[par * NB + k, sl] = s_v + off_p
                didx2g[par * NB + k, sl] = jnp.minimum(d_v + off_p, qmax)

    def issue_gathers(j, par, g):
        for k in range(NB):
            pltpu.async_copy(p_hbm.at[sidx2.at[par * NB + k]],
                             pbuf.at[pl.ds(par * K + k * EB, EB)], gsem)
            pltpu.async_copy(q_hbm.at[didx2g.at[par * NB + k]],
                             qbuf.at[pl.ds(par * K + k * EB, EB)], gsem)
        e0 = pl.multiple_of((s * ROWS_PT + j * NB) * EB, 8)
        pltpu.async_copy(b_hbm.at[pl.ds(e0, K),
                                  pl.ds(pl.multiple_of(g * TW, 32), TW)],
                         bbuf.at[pl.ds(par * K, K)], gsem)

    def drain_gathers(par):
        pltpu.make_async_copy(p_hbm.at[pl.ds(0, K)],
                              pbuf.at[pl.ds(par * K, K)], gsem).wait()
        pltpu.make_async_copy(q_hbm.at[pl.ds(0, K)],
                              qbuf.at[pl.ds(par * K, K)], gsem).wait()
        pltpu.make_async_copy(b_hbm.at[pl.ds(0, K), pl.ds(0, TW)],
                              bbuf.at[pl.ds(par * K, K)], gsem).wait()

    def issue_scatters(par):
        for k in range(NB):
            pltpu.async_copy(mbuf.at[pl.ds(par * K + k * EB, EB)],
                             agg_sh.at[didxr.at[par * NB + k]], ssem,
                             add=True)

    def drain_scatters(par):
        for k in range(NB):
            pltpu.make_async_copy(mbuf.at[pl.ds(par * K + k * EB, EB)],
                                  agg_sh.at[didxr.at[par * NB + k]],
                                  ssem).wait()

    def compute(par):
        @plsc.parallel_loop(par * K, (par + 1) * K, step=1, unroll=4)
        def _row(r):
            slo = pl.ds(0, GW)
            shi = pl.ds(GW, GW)
            cv = pbuf[r, slo] + qbuf[r, slo] + bbuf[r, slo]
            gv = pbuf[r, shi] + qbuf[r, shi] + bbuf[r, shi]
            denom = (1.0 + jnp.exp(-cv)) * (1.0 + jnp.exp(-gv))
            mbuf[r, slo] = cv / denom

    def pass_body(p):
        g = 2 * c + p
        off_p = g * N_ATOMS
        qmax = 4 * N_ATOMS - 1

        def zrow(r, carry):
            mbuf[r, pl.ds(0, GW)] = z16
            return carry

        lax.fori_loop(0, OP, zrow, 0)

        def zpiece(i, carry):
            pltpu.sync_copy(mbuf, agg_sh.at[pl.ds(n0 + i * OP, OP)])
            return carry

        lax.fori_loop(0, NOP, zpiece, 0)
        pltpu.sync_copy(mbuf.at[pl.ds(0, OPT)],
                        agg_sh.at[pl.ds(n0 + NOP * OP, OPT)])
        plsc.subcore_barrier()

        load_idx(0, 0)
        repack(0, off_p, qmax)
        issue_gathers(0, 0, g)

        def super_chunk(u, carry):
            for par in range(2):
                j = 2 * u + par

                if par == 0:
                    @pl.when(u > 0)
                    def _():
                        drain_scatters(1)
                else:
                    drain_scatters(0)

                @pl.when(j + 1 < CHUNKS)
                def _():
                    load_idx(j + 1, 1 - par)
                    repack(1 - par, off_p, qmax)
                    issue_gathers(j + 1, 1 - par, g)

                drain_gathers(par)
                compute(par)
                issue_scatters(par)
            return carry

        lax.fori_loop(0, CHUNKS // 2, super_chunk, 0)
        drain_scatters(1)
        plsc.subcore_barrier()

        def opiece(i, carry):
            r = pl.multiple_of(n0 + i * OP, 8)
            pltpu.sync_copy(
                agg_sh.at[pl.ds(r, OP)],
                out_hbm.at[pl.ds(pl.multiple_of(g * N_PAD + r, 8), OP)])
            return carry

        lax.fori_loop(0, NOP, opiece, 0)
        rt = pl.multiple_of(n0 + NOP * OP, 8)
        pltpu.sync_copy(
            agg_sh.at[pl.ds(rt, OPT)],
            out_hbm.at[pl.ds(pl.multiple_of(g * N_PAD + rt, 8), OPT)])
        plsc.subcore_barrier()

    for p in range(2):
        pass_body(p)


_conv_sc = pl.kernel(
    _conv_sc_body,
    out_type=jax.ShapeDtypeStruct((4 * N_PAD, GW), jnp.float32),
    mesh=plsc.VectorSubcoreMesh(core_axis_name="c", subcore_axis_name="s"),
    scratch_types=[
        pltpu.VMEM((2 * NB, EB), jnp.int32),
        pltpu.VMEM((2 * NB, EB), jnp.int32),
        pltpu.VMEM((2 * NB, EB), jnp.int32),
        pltpu.VMEM((2 * NB, EB), jnp.int32),
        pltpu.VMEM((2 * K, TW), jnp.float32),
        pltpu.VMEM((2 * K, TW), jnp.float32),
        pltpu.VMEM((2 * K, TW), jnp.float32),
        pltpu.VMEM((2 * K, GW), jnp.float32),
        pltpu.VMEM_SHARED((N_PAD, GW), jnp.float32),
        pltpu.SemaphoreType.DMA,
        pltpu.SemaphoreType.DMA,
    ],
    compiler_params=pltpu.CompilerParams(use_tc_tiling_on_sc=False),
)


def _layernorm(x):
    m = jnp.mean(x, axis=-1, keepdims=True)
    v = jnp.var(x, axis=-1, keepdims=True)
    return (x - m) / jnp.sqrt(v + 1e-5)


EBLK = 2048
NBLK = 2000


def _bterm_body(bb_ref, wb_ref, wi_ref, bc_ref, o_ref):
    y = jnp.dot(bb_ref[...], wb_ref[...], preferred_element_type=jnp.float32)
    bf = y * jax.nn.sigmoid(y)
    o_ref[...] = jnp.dot(bf, wi_ref[...],
                         preferred_element_type=jnp.float32) + bc_ref[...]


def _bterm(bond_bases, Wb, WBi, bci):
    wi = jnp.concatenate([WBi[g] for g in range(4)], 1)
    bc = jnp.concatenate([bci[g] for g in range(4)]).reshape(1, -1)
    return pl.pallas_call(
        _bterm_body,
        grid=(N_EDGES // NBLK,),
        in_specs=[
            pl.BlockSpec((NBLK, 31), lambda e: (e, 0)),
            pl.BlockSpec((31, FEA), lambda e: (0, 0)),
            pl.BlockSpec((FEA, 128), lambda e: (0, 0)),
            pl.BlockSpec((1, 128), lambda e: (0, 0)),
        ],
        out_specs=pl.BlockSpec((NBLK, 128), lambda e: (e, 0)),
        out_shape=jax.ShapeDtypeStruct((E_PAD, 128), jnp.float32),
    )(bond_bases, Wb, wi, bc)


def _embed_body(an_ref, emb_ref, x_ref):
    an = an_ref[0, 0, :]
    onehot = (an[:, None] == lax.broadcasted_iota(jnp.int32, (NBLK, 128), 1))
    x_ref[...] = jnp.dot(onehot.astype(jnp.float32), emb_ref[...],
                         preferred_element_type=jnp.float32)


def _embed(an3, emb_p):
    return pl.pallas_call(
        _embed_body,
        grid=(N_ATOMS // NBLK,),
        in_specs=[
            pl.BlockSpec((1, 1, NBLK), lambda b: (b, 0, 0)),
            pl.BlockSpec((128, FEA), lambda b: (0, 0)),
        ],
        out_specs=pl.BlockSpec((NBLK, FEA), lambda b: (b, 0)),
        out_shape=jax.ShapeDtypeStruct((N_ATOMS, FEA), jnp.float32),
    )(an3, emb_p)


def _proj_body(x_ref, wp_ref, wq_ref, p_ref, q_ref):
    x = x_ref[...]
    for g in range(4):
        p_ref[g] = jnp.dot(x, wp_ref[g], preferred_element_type=jnp.float32)
        q_ref[g] = jnp.dot(x, wq_ref[g], preferred_element_type=jnp.float32)


def _proj(x, WPi, WQi):
    P, Q = pl.pallas_call(
        _proj_body,
        grid=(N_ATOMS // NBLK,),
        in_specs=[
            pl.BlockSpec((NBLK, FEA), lambda b: (b, 0)),
            pl.BlockSpec((4, FEA, TW), lambda b: (0, 0, 0)),
            pl.BlockSpec((4, FEA, TW), lambda b: (0, 0, 0)),
        ],
        out_specs=[pl.BlockSpec((4, NBLK, TW), lambda b: (0, b, 0))] * 2,
        out_shape=[jax.ShapeDtypeStruct((4, N_ATOMS, TW), jnp.float32)] * 2,
    )(x, WPi, WQi)
    return P.reshape(4 * N_ATOMS, TW), Q.reshape(4 * N_ATOMS, TW)


def _update_body(x_ref, agg_ref, s_ref, b_ref, xo_ref):
    a = jnp.concatenate([agg_ref[g] for g in range(4)], axis=-1)
    xo_ref[...] = x_ref[...] + _layernorm(a) * s_ref[...] + b_ref[...]


def _update(x, agg4, ln_s_i, ln_b_i):
    return pl.pallas_call(
        _update_body,
        grid=(N_ATOMS // NBLK,),
        in_specs=[
            pl.BlockSpec((NBLK, FEA), lambda b: (b, 0)),
            pl.BlockSpec((4, NBLK, GW), lambda b: (0, b, 0)),
            pl.BlockSpec((1, FEA), lambda b: (0, 0)),
            pl.BlockSpec((1, FEA), lambda b: (0, 0)),
        ],
        out_specs=pl.BlockSpec((NBLK, FEA), lambda b: (b, 0)),
        out_shape=jax.ShapeDtypeStruct((N_ATOMS, FEA), jnp.float32),
    )(x, agg4, ln_s_i, ln_b_i)


def _readout_body(ow_ref, x_ref, wm_ref, bm_ref, wo_ref, bo_ref,
                  e_ref, gsum_ref, gcnt_ref):
    b = pl.program_id(0)

    @pl.when(b == 0)
    def _():
        gsum_ref[...] = jnp.zeros_like(gsum_ref)
        gcnt_ref[...] = jnp.zeros_like(gcnt_ref)

    ow = ow_ref[0, 0, :]
    onehot = (ow[:, None] == lax.broadcasted_iota(
        jnp.int32, (NBLK, N_GRAPHS), 1)).astype(jnp.float32)
    gsum_ref[...] += jnp.dot(onehot.T, x_ref[...],
                             preferred_element_type=jnp.float32)
    gcnt_ref[...] += jnp.sum(onehot, axis=0, keepdims=True)

    @pl.when(b == N_ATOMS // NBLK - 1)
    def _():
        h = gsum_ref[...] / jnp.maximum(gcnt_ref[...], 1.0).T
        for j in range(3):
            y = jnp.dot(h, wm_ref[j], preferred_element_type=jnp.float32) \
                + bm_ref[j]
            h = y * jax.nn.sigmoid(y)
        e_ref[...] = jnp.dot(h, wo_ref[...],
                             preferred_element_type=jnp.float32) + bo_ref[...]


def _readout(ow3, x, Wm, bm2, Wout, bout2):
    e, _, _ = pl.pallas_call(
        _readout_body,
        grid=(N_ATOMS // NBLK,),
        in_specs=[
            pl.BlockSpec((1, 1, NBLK), lambda b: (b, 0, 0)),
            pl.BlockSpec((NBLK, FEA), lambda b: (b, 0)),
            pl.BlockSpec((3, FEA, FEA), lambda b: (0, 0, 0)),
            pl.BlockSpec((3, 1, FEA), lambda b: (0, 0, 0)),
            pl.BlockSpec((FEA, 1), lambda b: (0, 0)),
            pl.BlockSpec((1, 1), lambda b: (0, 0)),
        ],
        out_specs=[
            pl.BlockSpec((N_GRAPHS, 1), lambda b: (0, 0)),
            pl.BlockSpec((N_GRAPHS, FEA), lambda b: (0, 0)),
            pl.BlockSpec((1, N_GRAPHS), lambda b: (0, 0)),
        ],
        out_shape=[
            jax.ShapeDtypeStruct((N_GRAPHS, 1), jnp.float32),
            jax.ShapeDtypeStruct((N_GRAPHS, FEA), jnp.float32),
            jax.ShapeDtypeStruct((1, N_GRAPHS), jnp.float32),
        ],
    )(ow3, x, Wm, bm2, Wout, bout2)
    return e[:, 0]


def kernel(atomic_numbers, bond_bases_ag, batched_atom_graph, atom_owners,
           emb, Wb, W1, b1, Wg, bg, ln_s, ln_b, Wm, bm, Wout, bout):
    n = N_ATOMS
    e = N_EDGES

    src = batched_atom_graph[:, 0].astype(jnp.int32)
    dst = batched_atom_graph[:, 1].astype(jnp.int32)
    pad = E_PAD - e
    src_p = jnp.concatenate(
        [src, jnp.zeros((pad,), jnp.int32)]).reshape(E_PAD // EB, EB)
    dst_p = jnp.concatenate(
        [dst, jnp.full((pad,), N_ATOMS, jnp.int32)]).reshape(E_PAD // EB, EB)

    def pack(lo):
        return jnp.stack([
            jnp.concatenate([W1[i][lo:lo + FEA, g * GW:(g + 1) * GW],
                             Wg[i][lo:lo + FEA, g * GW:(g + 1) * GW]], 1)
            for i in range(N_CONV) for g in range(4)]).reshape(
                N_CONV, 4, FEA, TW)

    WP = pack(0)
    WQ = pack(FEA)
    WB = pack(2 * FEA)
    bcat = jnp.stack([
        jnp.concatenate([b1[i][g * GW:(g + 1) * GW],
                         bg[i][g * GW:(g + 1) * GW]])
        for i in range(N_CONV) for g in range(4)]).reshape(N_CONV, 4, TW)

    an3 = atomic_numbers.astype(jnp.int32).reshape(
        N_ATOMS // NBLK, 1, NBLK)
    emb_p = jnp.concatenate(
        [emb, jnp.zeros((128 - emb.shape[0], FEA), jnp.float32)], 0)
    x = _embed(an3, emb_p)

    for i in range(N_CONV):
        Bi = _bterm(bond_bases_ag, Wb, WB[i], bcat[i])
        P, Q = _proj(x, WP[i], WQ[i])
        agg4 = _conv_sc(src_p, dst_p, P, Q, Bi)
        agg4 = agg4.reshape(4, N_PAD, GW)
        x = _update(x, agg4, ln_s[i].reshape(1, FEA),
                    ln_b[i].reshape(1, FEA))

    ow3 = atom_owners.astype(jnp.int32).reshape(N_ATOMS // NBLK, 1, NBLK)
    return _readout(ow3, x, Wm, bm.reshape(3, 1, FEA), Wout,
                    bout.reshape(1, 1))
